# baseline (speedup 1.0000x reference)
import functools

import jax
import jax.numpy as jnp
from jax import lax
from jax.experimental import pallas as pl
from jax.experimental.pallas import tpu as pltpu
from jax.experimental.pallas import tpu_sc as plsc

_LANES = 128
_NBUF = 4


def _project_table(pitch_embed, W, b):
    V, H = pitch_embed.shape
    O = W.shape[0]

    def body(e_ref, w_ref, b_ref, o_ref):
        res = lax.dot_general(
            e_ref[...], w_ref[...],
            dimension_numbers=(((1,), (1,)), ((), ())),
            preferred_element_type=jnp.float32,
        ) + b_ref[...]
        o_ref[...] = jnp.concatenate(
            [res, jnp.zeros((V, _LANES - O), jnp.float32)], axis=1)

    return pl.pallas_call(
        body,
        out_shape=jax.ShapeDtypeStruct((V, _LANES), jnp.float32),
    )(pitch_embed, W, b.reshape(1, O))


@functools.lru_cache(maxsize=None)
def _make_sc_gather(V, O, NW, NC, C):
    mesh = plsc.VectorSubcoreMesh(core_axis_name="c", subcore_axis_name="s")

    @functools.partial(
        pl.kernel,
        out_type=jax.ShapeDtypeStruct((NW * C * 128, O), jnp.float32),
        mesh=mesh,
        scratch_types=[
            pltpu.VMEM((C, 128), jnp.int32),
            pltpu.VMEM((_NBUF, 128, _LANES), jnp.float32),
            pltpu.VMEM_SHARED((V, _LANES), jnp.float32),
        ] + [pltpu.SemaphoreType.DMA] * (2 * _NBUF),
        compiler_params=pltpu.CompilerParams(use_tc_tiling_on_sc=False),
    )
    def k(table_hbm, idx_hbm, out_hbm, idx_v, rows_v, table_sp, *sems):
        gsem = sems[:_NBUF]
        ssem = sems[_NBUF:]
        sid = lax.axis_index("s")
        wid = sid * NC + lax.axis_index("c")
        base = wid * C * 128
        @pl.when(sid == 0)
        def _():
            pltpu.sync_copy(table_hbm, table_sp)
        pltpu.sync_copy(idx_hbm.at[pl.ds(wid * C, C)], idx_v)
        plsc.subcore_barrier()
        gathers = [None] * _NBUF
        stores = [None] * _NBUF
        for j in range(min(_NBUF, C)):
            gathers[j] = pltpu.async_copy(
                table_sp.at[idx_v.at[j]], rows_v.at[j], gsem[j])
        for j in range(C):
            buf = j % _NBUF
            gathers[buf].wait()
            stores[buf] = pltpu.async_copy(
                rows_v.at[buf, :, pl.ds(0, O)],
                out_hbm.at[pl.ds(base + j * 128, 128)], ssem[buf])
            nj = j + _NBUF
            if nj < C:
                stores[buf].wait()
                stores[buf] = None
                gathers[buf] = pltpu.async_copy(
                    table_sp.at[idx_v.at[nj]], rows_v.at[buf], gsem[buf])
        for st in stores:
            if st is not None:
                st.wait()

    return k


def kernel(mel, pitch_embed, W, b):
    B, T = mel.shape
    V, H = pitch_embed.shape
    O = W.shape[0]

    info = plsc.get_sparse_core_info()
    NC, NS = info.num_cores, info.num_subcores
    NW = NC * NS
    tokens = B * T
    assert tokens % (NW * 128) == 0
    C = tokens // (NW * 128)

    proj = _project_table(pitch_embed, W, b)
    idx = mel.reshape(tokens // 128, 128).astype(jnp.int32)
    out = _make_sc_gather(V, O, NW, NC, C)(proj, idx)
    return out.reshape(B, T, O)


# speedup vs baseline: 1.0647x; 1.0647x over previous
import functools

import jax
import jax.numpy as jnp
from jax import lax
from jax.experimental import pallas as pl
from jax.experimental.pallas import tpu as pltpu
from jax.experimental.pallas import tpu_sc as plsc

_LANES = 128
_NBUF = 4


def _project_table(pitch_embed, W, b):
    V, H = pitch_embed.shape
    O = W.shape[0]

    def body(e_ref, w_ref, b_ref, o_ref):
        res = lax.dot_general(
            e_ref[...], w_ref[...],
            dimension_numbers=(((1,), (1,)), ((), ())),
            preferred_element_type=jnp.float32,
        ) + b_ref[...]
        o_ref[...] = jnp.concatenate(
            [res, jnp.zeros((V, _LANES - O), jnp.float32)], axis=1)

    return pl.pallas_call(
        body,
        out_shape=jax.ShapeDtypeStruct((V, _LANES), jnp.float32),
    )(pitch_embed, W, b.reshape(1, O))


@functools.lru_cache(maxsize=None)
def _make_sc_gather(V, NW, NC, C):
    mesh = plsc.VectorSubcoreMesh(core_axis_name="c", subcore_axis_name="s")

    @functools.partial(
        pl.kernel,
        out_type=jax.ShapeDtypeStruct((NW * C * 128, _LANES), jnp.float32),
        mesh=mesh,
        scratch_types=[
            pltpu.VMEM((C, 128), jnp.int32),
            pltpu.VMEM((_NBUF, 128, _LANES), jnp.float32),
            pltpu.VMEM_SHARED((V, _LANES), jnp.float32),
        ] + [pltpu.SemaphoreType.DMA] * (2 * _NBUF),
    )
    def k(table_hbm, idx_hbm, out_hbm, idx_v, rows_v, table_sp, *sems):
        gsem = sems[:_NBUF]
        ssem = sems[_NBUF:]
        sid = lax.axis_index("s")
        wid = sid * NC + lax.axis_index("c")
        base = wid * C * 128
        @pl.when(sid == 0)
        def _():
            pltpu.sync_copy(table_hbm, table_sp)
        pltpu.sync_copy(idx_hbm.at[pl.ds(wid * C, C)], idx_v)
        plsc.subcore_barrier()
        gathers = [None] * _NBUF
        stores = [None] * _NBUF
        for j in range(min(_NBUF, C)):
            gathers[j] = pltpu.async_copy(
                table_sp.at[idx_v.at[j]], rows_v.at[j], gsem[j])
        for j in range(C):
            buf = j % _NBUF
            gathers[buf].wait()
            stores[buf] = pltpu.async_copy(
                rows_v.at[buf],
                out_hbm.at[pl.ds(base + j * 128, 128)], ssem[buf])
            nj = j + _NBUF
            if nj < C:
                stores[buf].wait()
                stores[buf] = None
                gathers[buf] = pltpu.async_copy(
                    table_sp.at[idx_v.at[nj]], rows_v.at[buf], gsem[buf])
        for st in stores:
            if st is not None:
                st.wait()

    return k


def _unpad_body(in_ref, o_ref):
    o_ref[...] = in_ref[:, : o_ref.shape[-1]]


def _unpad(padded, O, nblk):
    N = padded.shape[0]
    return pl.pallas_call(
        _unpad_body,
        grid=(N // nblk,),
        in_specs=[pl.BlockSpec((nblk, _LANES), lambda i: (i, 0))],
        out_specs=pl.BlockSpec((nblk, O), lambda i: (i, 0)),
        out_shape=jax.ShapeDtypeStruct((N, O), jnp.float32),
    )(padded)


def kernel(mel, pitch_embed, W, b):
    B, T = mel.shape
    V, H = pitch_embed.shape
    O = W.shape[0]

    info = plsc.get_sparse_core_info()
    NC, NS = info.num_cores, info.num_subcores
    NW = NC * NS
    tokens = B * T
    assert tokens % (NW * 128) == 0
    C = tokens // (NW * 128)

    proj = _project_table(pitch_embed, W, b)
    idx = mel.reshape(tokens // 128, 128).astype(jnp.int32)
    padded = _make_sc_gather(V, NW, NC, C)(proj, idx)
    out = _unpad(padded, O, 4096)
    return out.reshape(B, T, O)


# speedup vs baseline: 1.0832x; 1.0174x over previous
import functools

import jax
import jax.numpy as jnp
from jax import lax
from jax.experimental import pallas as pl
from jax.experimental.pallas import tpu as pltpu
from jax.experimental.pallas import tpu_sc as plsc

_LANES = 128
_NBUF = 4


def _project_table(pitch_embed, W, b):
    V, H = pitch_embed.shape
    O = W.shape[0]

    def body(e_ref, w_ref, b_ref, o_ref):
        res = lax.dot_general(
            e_ref[...], w_ref[...],
            dimension_numbers=(((1,), (1,)), ((), ())),
            preferred_element_type=jnp.float32,
        ) + b_ref[...]
        o_ref[...] = jnp.concatenate(
            [res, jnp.zeros((V, _LANES - O), jnp.float32)], axis=1)

    return pl.pallas_call(
        body,
        out_shape=jax.ShapeDtypeStruct((V, _LANES), jnp.float32),
    )(pitch_embed, W, b.reshape(1, O))


@functools.lru_cache(maxsize=None)
def _make_sc_gather(V, NW, NC, C):
    mesh = plsc.VectorSubcoreMesh(core_axis_name="c", subcore_axis_name="s")

    @functools.partial(
        pl.kernel,
        out_type=jax.ShapeDtypeStruct((NW * C * 128, _LANES), jnp.float32),
        mesh=mesh,
        scratch_types=[
            pltpu.VMEM((C, 128), jnp.int32),
            pltpu.VMEM((_NBUF, 128, _LANES), jnp.float32),
            pltpu.VMEM_SHARED((V, _LANES), jnp.float32),
        ] + [pltpu.SemaphoreType.DMA] * (2 * _NBUF),
    )
    def k(table_hbm, idx_hbm, out_hbm, idx_v, rows_v, table_sp, *sems):
        gsem = sems[:_NBUF]
        ssem = sems[_NBUF:]
        sid = lax.axis_index("s")
        wid = sid * NC + lax.axis_index("c")
        base = wid * C * 128
        @pl.when(sid == 0)
        def _():
            pltpu.sync_copy(table_hbm, table_sp)
        pltpu.sync_copy(idx_hbm.at[pl.ds(wid * C, C)], idx_v)
        plsc.subcore_barrier()
        gathers = [None] * _NBUF
        stores = [None] * _NBUF
        for j in range(min(_NBUF, C)):
            gathers[j] = pltpu.async_copy(
                table_sp.at[idx_v.at[j]], rows_v.at[j], gsem[j])
        for j in range(C):
            buf = j % _NBUF
            gathers[buf].wait()
            stores[buf] = pltpu.async_copy(
                rows_v.at[buf],
                out_hbm.at[pl.ds(base + j * 128, 128)], ssem[buf])
            nj = j + _NBUF
            if nj < C:
                stores[buf].wait()
                stores[buf] = None
                gathers[buf] = pltpu.async_copy(
                    table_sp.at[idx_v.at[nj]], rows_v.at[buf], gsem[buf])
        for st in stores:
            if st is not None:
                st.wait()

    return k


def _unpad_body(in_ref, o_ref):
    o_ref[...] = in_ref[:, : o_ref.shape[-1]]


def _unpad(padded, O, nblk):
    N = padded.shape[0]
    return pl.pallas_call(
        _unpad_body,
        grid=(N // nblk,),
        in_specs=[pl.BlockSpec((nblk, _LANES), lambda i: (i, 0))],
        out_specs=pl.BlockSpec((nblk, O), lambda i: (i, 0)),
        out_shape=jax.ShapeDtypeStruct((N, O), jnp.float32),
    )(padded)


def kernel(mel, pitch_embed, W, b):
    B, T = mel.shape
    V, H = pitch_embed.shape
    O = W.shape[0]

    info = plsc.get_sparse_core_info()
    NC, NS = info.num_cores, info.num_subcores
    NW = NC * NS
    tokens = B * T
    assert tokens % (NW * 128) == 0
    C = tokens // (NW * 128)

    proj = _project_table(pitch_embed, W, b)
    idx = mel.reshape(tokens // 128, 128).astype(jnp.int32)
    padded = _make_sc_gather(V, NW, NC, C)(proj, idx)
    one = b[0] - b[0] + jnp.float32(1.0)
    return padded[:, :O].reshape(B, T, O) * one


# speedup vs baseline: 1.3382x; 1.2354x over previous
import functools

import jax
import jax.numpy as jnp
from jax import lax
from jax.experimental import pallas as pl
from jax.experimental.pallas import tpu as pltpu
from jax.experimental.pallas import tpu_sc as plsc

_LANES = 128
_NBUF = 4


def _project_table(pitch_embed, W, b):
    V, H = pitch_embed.shape
    O = W.shape[0]

    def body(e_ref, w_ref, b_ref, o_ref):
        res = lax.dot_general(
            e_ref[...], w_ref[...],
            dimension_numbers=(((1,), (1,)), ((), ())),
            preferred_element_type=jnp.float32,
        ) + b_ref[...]
        o_ref[...] = jnp.concatenate(
            [res, jnp.zeros((V, _LANES - O), jnp.float32)], axis=1)

    return pl.pallas_call(
        body,
        out_shape=jax.ShapeDtypeStruct((V, _LANES), jnp.float32),
    )(pitch_embed, W, b.reshape(1, O))


@functools.lru_cache(maxsize=None)
def _make_sc_gather(V, NW, NC, C):
    mesh = plsc.VectorSubcoreMesh(core_axis_name="c", subcore_axis_name="s")

    @functools.partial(
        pl.kernel,
        out_type=jax.ShapeDtypeStruct((NW * C * 128, _LANES), jnp.float32),
        mesh=mesh,
        scratch_types=[
            pltpu.VMEM((C, 128), jnp.int32),
            pltpu.VMEM((_NBUF, 128, _LANES), jnp.float32),
            pltpu.VMEM_SHARED((V, _LANES), jnp.float32),
        ] + [pltpu.SemaphoreType.DMA] * (2 * _NBUF),
    )
    def k(table_hbm, idx_hbm, out_hbm, idx_v, rows_v, table_sp, *sems):
        gsem = sems[:_NBUF]
        ssem = sems[_NBUF:]
        sid = lax.axis_index("s")
        wid = sid * NC + lax.axis_index("c")
        base = wid * C * 128
        @pl.when(sid == 0)
        def _():
            pltpu.sync_copy(table_hbm, table_sp)
        pltpu.sync_copy(idx_hbm.at[pl.ds(wid * C, C)], idx_v)
        plsc.subcore_barrier()
        gathers = [None] * _NBUF
        stores = [None] * _NBUF
        for j in range(min(_NBUF, C)):
            gathers[j] = pltpu.async_copy(
                table_sp.at[idx_v.at[j]], rows_v.at[j], gsem[j])
        for j in range(C):
            buf = j % _NBUF
            gathers[buf].wait()
            stores[buf] = pltpu.async_copy(
                rows_v.at[buf],
                out_hbm.at[pl.ds(base + j * 128, 128)], ssem[buf])
            nj = j + _NBUF
            if nj < C:
                stores[buf].wait()
                stores[buf] = None
                gathers[buf] = pltpu.async_copy(
                    table_sp.at[idx_v.at[nj]], rows_v.at[buf], gsem[buf])
        for st in stores:
            if st is not None:
                st.wait()

    return k


def kernel(mel, pitch_embed, W, b):
    B, T = mel.shape
    V, H = pitch_embed.shape
    O = W.shape[0]

    info = plsc.get_sparse_core_info()
    NC, NS = info.num_cores, info.num_subcores
    NW = NC * NS
    tokens = B * T
    assert tokens % (NW * 128) == 0
    C = tokens // (NW * 128)

    proj = _project_table(pitch_embed, W, b)
    idx = mel.reshape(tokens // 128, 128).astype(jnp.int32)
    padded = _make_sc_gather(V, NW, NC, C)(proj, idx)
    return padded[:, :O].reshape(B, T, O)
